# baseline (speedup 1.0000x reference)
import jax
import jax.numpy as jnp
from jax import lax
from jax.experimental import pallas as pl
from jax.experimental.pallas import tpu as pltpu
from jax.experimental.pallas import tpu_sc as plsc

_COL = 42
_ROWS = 16384
_NCOLS = 100
_INFO = plsc.get_sparse_core_info()
_NC = _INFO.num_cores
_NS = _INFO.num_subcores
_NW = _NC * _NS
_RPW = _ROWS // _NW
_CHUNK = 128


def _sc_select_column(x_flat):
    mesh = plsc.VectorSubcoreMesh(core_axis_name="c", subcore_axis_name="s")

    @pl.kernel(
        out_type=jax.ShapeDtypeStruct((_ROWS,), jnp.float32),
        mesh=mesh,
        scratch_types=[
            pltpu.VMEM((_RPW,), jnp.int32),
            pltpu.VMEM((_RPW,), jnp.float32),
            pltpu.SemaphoreType.DMA,
        ],
    )
    def k(x_hbm, out_hbm, idx_v, val_v, sem):
        wid = lax.axis_index("s") * _NC + lax.axis_index("c")
        base = wid * _RPW
        for j in range(_RPW // 16):
            idx_v[pl.ds(j * 16, 16)] = (
                lax.iota(jnp.int32, 16) * _NCOLS + (base + j * 16) * _NCOLS + _COL
            )
        copies = [
            pltpu.async_copy(
                x_hbm.at[idx_v.at[pl.ds(t * _CHUNK, _CHUNK)]],
                val_v.at[pl.ds(t * _CHUNK, _CHUNK)],
                sem,
            )
            for t in range(_RPW // _CHUNK)
        ]
        for c in copies:
            c.wait()
        pltpu.sync_copy(val_v, out_hbm.at[pl.ds(base, _RPW)])

    return k(x_flat)


def kernel(x):
    if x.ndim == 1:
        x = x[None, :]
    x_flat = x.astype(jnp.float32).reshape(-1)
    return _sc_select_column(x_flat).reshape(_ROWS, 1)


# speedup vs baseline: 1.3731x; 1.3731x over previous
import jax
import jax.numpy as jnp
from jax import lax
from jax.experimental import pallas as pl
from jax.experimental.pallas import tpu as pltpu
from jax.experimental.pallas import tpu_sc as plsc

_COL = 42
_ROWS = 16384
_NCOLS = 100
_INFO = plsc.get_sparse_core_info()
_NC = _INFO.num_cores
_NS = _INFO.num_subcores
_NW = _NC * _NS
_RPW = _ROWS // _NW


def _sc_select_column(x):
    mesh = plsc.VectorSubcoreMesh(core_axis_name="c", subcore_axis_name="s")

    @pl.kernel(
        out_type=jax.ShapeDtypeStruct((_ROWS,), jnp.float32),
        mesh=mesh,
        compiler_params=pltpu.CompilerParams(needs_layout_passes=False),
        scratch_types=[
            pltpu.VMEM((_RPW, _NCOLS), jnp.float32),
            pltpu.VMEM((_RPW,), jnp.float32),
        ],
    )
    def k(x_hbm, out_hbm, slab_v, out_v):
        wid = lax.axis_index("s") * _NC + lax.axis_index("c")
        base = wid * _RPW
        pltpu.sync_copy(x_hbm.at[pl.ds(base, _RPW)], slab_v)
        lane0 = lax.iota(jnp.int32, 16) == 0
        for r in range(_RPW):
            v = slab_v[r, pl.ds(_COL, 16)]
            plsc.store_scatter(out_v, [jnp.full((16,), r, jnp.int32)], v, mask=lane0)
        pltpu.sync_copy(out_v, out_hbm.at[pl.ds(base, _RPW)])

    return k(x)


def kernel(x):
    if x.ndim == 1:
        x = x[None, :]
    return _sc_select_column(x.astype(jnp.float32)).reshape(_ROWS, 1)


# speedup vs baseline: 1.4593x; 1.0628x over previous
import jax
import jax.numpy as jnp
from jax import lax
from jax.experimental import pallas as pl
from jax.experimental.pallas import tpu as pltpu
from jax.experimental.pallas import tpu_sc as plsc

_COL = 42
_ROWS = 16384
_NCOLS = 100
_INFO = plsc.get_sparse_core_info()
_NC = _INFO.num_cores
_NS = _INFO.num_subcores
_NW = _NC * _NS
_RPW = _ROWS // _NW


def _sc_select_column(x):
    mesh = plsc.VectorSubcoreMesh(core_axis_name="c", subcore_axis_name="s")

    @pl.kernel(
        out_type=jax.ShapeDtypeStruct((_ROWS,), jnp.float32),
        mesh=mesh,
        compiler_params=pltpu.CompilerParams(
            needs_layout_passes=False,
            disable_bounds_checks=True,
            disable_semaphore_checks=True,
            skip_device_barrier=True,
        ),
        scratch_types=[
            pltpu.VMEM((_RPW, _NCOLS), jnp.float32),
            pltpu.VMEM((_RPW,), jnp.float32),
        ],
    )
    def k(x_hbm, out_hbm, slab_v, out_v):
        wid = lax.axis_index("s") * _NC + lax.axis_index("c")
        base = wid * _RPW
        pltpu.sync_copy(x_hbm.at[pl.ds(base, _RPW)], slab_v)
        cols = jnp.full((16,), _COL, jnp.int32)
        for j in range(_RPW // 16):
            rows = lax.iota(jnp.int32, 16) + j * 16
            v = plsc.load_gather(slab_v, [rows, cols])
            out_v[pl.ds(j * 16, 16)] = v
        pltpu.sync_copy(out_v, out_hbm.at[pl.ds(base, _RPW)])

    return k(x)


def kernel(x):
    if x.ndim == 1:
        x = x[None, :]
    return _sc_select_column(x.astype(jnp.float32)).reshape(_ROWS, 1)
